# Initial kernel scaffold; baseline (speedup 1.0000x reference)
import jax
import jax.numpy as jnp
from jax.experimental import pallas as pl


def kernel(roi_features, union_features, rel_pair_idx, up_W, up_b, objU_W, objU_b, edgeU_W, edgeU_b, node_Wih, node_bih, node_Whh, node_bhh, edge_Wih, edge_bih, edge_Whh, edge_bhh, subw_W, subw_b, objw_W, objw_b, outw_W, outw_b, inw_W, inw_b, objfc_W, objfc_b, relfc_W, relfc_b, freq_table):
    raise NotImplementedError("write your pallas kernel here")


# speedup vs baseline: 1.1388x; 1.1388x over previous
import functools

import jax
import jax.numpy as jnp
from jax import lax
from jax.experimental import pallas as pl
from jax.experimental.pallas import tpu as pltpu
from jax.experimental.pallas import tpu_sc as plsc

H = 512
NUM_OBJ = 151
NUM_REL = 51
STEPS = 3


def _mm_kernel(a_ref, b_ref, bias_ref, o_ref):
    o_ref[...] = jnp.dot(a_ref[...], b_ref[...],
                         preferred_element_type=jnp.float32) + bias_ref[...]


def _matmul_bias(a, b, bias, *, bm=256, bn=512):
    M, K = a.shape
    _, N = b.shape
    return pl.pallas_call(
        _mm_kernel,
        grid=(M // bm, N // bn),
        in_specs=[
            pl.BlockSpec((bm, K), lambda i, j: (i, 0)),
            pl.BlockSpec((K, bn), lambda i, j: (0, j)),
            pl.BlockSpec((1, bn), lambda i, j: (0, j)),
        ],
        out_specs=pl.BlockSpec((bm, bn), lambda i, j: (i, j)),
        out_shape=jax.ShapeDtypeStruct((M, N), jnp.float32),
        compiler_params=pltpu.CompilerParams(
            dimension_semantics=("parallel", "parallel")),
    )(a, b, bias.reshape(1, N))


def _gru0_tail(acc, b1, relu, wih_ref, bih_ref, bhh_ref, o_ref):
    rep = acc + b1
    if relu:
        rep = jnp.maximum(rep, 0.0)
    gi = jnp.dot(rep, wih_ref[...],
                 preferred_element_type=jnp.float32) + bih_ref[...]
    bhh = bhh_ref[...]
    r = jax.nn.sigmoid(gi[:, :H] + bhh[:, :H])
    z = jax.nn.sigmoid(gi[:, H:2 * H] + bhh[:, H:2 * H])
    n = jnp.tanh(gi[:, 2 * H:] + r * bhh[:, 2 * H:])
    o_ref[...] = (1.0 - z) * n


def _mm_gru0_kernel(a_ref, w1_ref, b1_ref, wih_ref, bih_ref, bhh_ref, o_ref,
                    acc_ref, *, nk, relu):
    k = pl.program_id(1)

    @pl.when(k == 0)
    def _():
        acc_ref[...] = jnp.zeros_like(acc_ref)

    acc_ref[...] += jnp.dot(a_ref[...], w1_ref[...],
                            preferred_element_type=jnp.float32)

    @pl.when(k == nk - 1)
    def _():
        _gru0_tail(acc_ref[...], b1_ref[...], relu, wih_ref, bih_ref,
                   bhh_ref, o_ref)


def _mm_gru0(a, w1, b1, wih, bih, bhh, *, relu, bm=256, bk=512):
    M, K = a.shape
    nk = K // bk
    return pl.pallas_call(
        functools.partial(_mm_gru0_kernel, nk=nk, relu=relu),
        grid=(M // bm, nk),
        in_specs=[
            pl.BlockSpec((bm, bk), lambda i, k: (i, k)),
            pl.BlockSpec((bk, H), lambda i, k: (k, 0)),
            pl.BlockSpec((1, H), lambda i, k: (0, 0)),
            pl.BlockSpec((H, 3 * H), lambda i, k: (0, 0)),
            pl.BlockSpec((1, 3 * H), lambda i, k: (0, 0)),
            pl.BlockSpec((1, 3 * H), lambda i, k: (0, 0)),
        ],
        out_specs=pl.BlockSpec((bm, H), lambda i, k: (i, 0)),
        out_shape=jax.ShapeDtypeStruct((M, H), jnp.float32),
        scratch_shapes=[pltpu.VMEM((bm, H), jnp.float32)],
        compiler_params=pltpu.CompilerParams(
            dimension_semantics=("parallel", "arbitrary")),
    )(a, w1, b1.reshape(1, H), wih, bih.reshape(1, 3 * H),
      bhh.reshape(1, 3 * H))


def _edge_step_kernel(sv_ref, ov_ref, e_ref, wa_ref, ba_ref, wb_ref, bb_ref,
                      wih_ref, bih_ref, whh_ref, bhh_ref, ne_ref, msmo_ref):
    sv = sv_ref[...]
    ov = ov_ref[...]
    e = e_ref[...]
    wa = wa_ref[...]
    wb = wb_ref[...]
    xa = jnp.concatenate([sv, e], axis=1)
    xb = jnp.concatenate([ov, e], axis=1)
    ga = jnp.dot(xa, wa, preferred_element_type=jnp.float32) + ba_ref[...]
    gb = jnp.dot(xb, wb, preferred_element_type=jnp.float32) + bb_ref[...]
    ws = jax.nn.sigmoid(ga[:, 0:1])
    w_out = jax.nn.sigmoid(ga[:, 1:2])
    wo = jax.nn.sigmoid(gb[:, 0:1])
    w_in = jax.nn.sigmoid(gb[:, 1:2])
    x = ws * sv + wo * ov
    gi = jnp.dot(x, wih_ref[...],
                 preferred_element_type=jnp.float32) + bih_ref[...]
    gh = jnp.dot(e, whh_ref[...],
                 preferred_element_type=jnp.float32) + bhh_ref[...]
    r = jax.nn.sigmoid(gi[:, :H] + gh[:, :H])
    z = jax.nn.sigmoid(gi[:, H:2 * H] + gh[:, H:2 * H])
    n = jnp.tanh(gi[:, 2 * H:] + r * gh[:, 2 * H:])
    ne_ref[...] = (1.0 - z) * n + z * e
    msmo_ref[0] = w_out * e
    msmo_ref[1] = w_in * e


def _edge_step(sv, ov, e, wa, ba, wb, bb, wih, bih, whh, bhh, *, bm=256):
    E = e.shape[0]
    row = lambda i: (i, 0)
    cst = lambda i: (0, 0)
    out_sds = jax.ShapeDtypeStruct((E, H), jnp.float32)
    return pl.pallas_call(
        _edge_step_kernel,
        grid=(E // bm,),
        in_specs=[
            pl.BlockSpec((bm, H), row),
            pl.BlockSpec((bm, H), row),
            pl.BlockSpec((bm, H), row),
            pl.BlockSpec((2 * H, 2), cst),
            pl.BlockSpec((1, 2), cst),
            pl.BlockSpec((2 * H, 2), cst),
            pl.BlockSpec((1, 2), cst),
            pl.BlockSpec((H, 3 * H), cst),
            pl.BlockSpec((1, 3 * H), cst),
            pl.BlockSpec((H, 3 * H), cst),
            pl.BlockSpec((1, 3 * H), cst),
        ],
        out_specs=[
            pl.BlockSpec((bm, H), row),
            pl.BlockSpec((2, bm, H), lambda i: (0, i, 0)),
        ],
        out_shape=[out_sds,
                   jax.ShapeDtypeStruct((2, E, H), jnp.float32)],
        compiler_params=pltpu.CompilerParams(
            dimension_semantics=("parallel",)),
    )(sv, ov, e, wa, ba, wb, bb, wih, bih.reshape(1, 3 * H), whh,
      bhh.reshape(1, 3 * H))


def _node_gru_kernel(x_ref, h_ref, wih_ref, bih_ref, whh_ref, bhh_ref, o_ref):
    h = h_ref[...]
    gi = jnp.dot(x_ref[...], wih_ref[...],
                 preferred_element_type=jnp.float32) + bih_ref[...]
    gh = jnp.dot(h, whh_ref[...],
                 preferred_element_type=jnp.float32) + bhh_ref[...]
    r = jax.nn.sigmoid(gi[:, :H] + gh[:, :H])
    z = jax.nn.sigmoid(gi[:, H:2 * H] + gh[:, H:2 * H])
    n = jnp.tanh(gi[:, 2 * H:] + r * gh[:, 2 * H:])
    o_ref[...] = (1.0 - z) * n + z * h


def _node_gru(x, h, wih, bih, whh, bhh, *, bm=256):
    M = x.shape[0]
    row = lambda i: (i, 0)
    cst = lambda i: (0, 0)
    return pl.pallas_call(
        _node_gru_kernel,
        grid=(M // bm,),
        in_specs=[
            pl.BlockSpec((bm, H), row),
            pl.BlockSpec((bm, H), row),
            pl.BlockSpec((H, 3 * H), cst),
            pl.BlockSpec((1, 3 * H), cst),
            pl.BlockSpec((H, 3 * H), cst),
            pl.BlockSpec((1, 3 * H), cst),
        ],
        out_specs=pl.BlockSpec((bm, H), row),
        out_shape=jax.ShapeDtypeStruct((M, H), jnp.float32),
        compiler_params=pltpu.CompilerParams(
            dimension_semantics=("parallel",)),
    )(x, h, wih, bih.reshape(1, 3 * H), whh, bhh.reshape(1, 3 * H))


def _obj_head_kernel(v_ref, w_ref, b_ref, d_ref, p_ref):
    d = jnp.dot(v_ref[...], w_ref[...],
                preferred_element_type=jnp.float32) + b_ref[...]
    d_ref[...] = d
    m = jnp.max(d, axis=1, keepdims=True)
    idx = lax.broadcasted_iota(jnp.int32, d.shape, 1)
    p = jnp.min(jnp.where(d == m, idx, NUM_OBJ), axis=1, keepdims=True)
    p_ref[...] = jnp.broadcast_to(p, p_ref.shape)


def _obj_head(v, w, b, *, bm=512):
    M = v.shape[0]
    return pl.pallas_call(
        _obj_head_kernel,
        grid=(M // bm,),
        in_specs=[
            pl.BlockSpec((bm, H), lambda i: (i, 0)),
            pl.BlockSpec((H, NUM_OBJ), lambda i: (0, 0)),
            pl.BlockSpec((1, NUM_OBJ), lambda i: (0, 0)),
        ],
        out_specs=[
            pl.BlockSpec((bm, NUM_OBJ), lambda i: (i, 0)),
            pl.BlockSpec((bm, 128), lambda i: (i, 0)),
        ],
        out_shape=[
            jax.ShapeDtypeStruct((M, NUM_OBJ), jnp.float32),
            jax.ShapeDtypeStruct((M, 128), jnp.int32),
        ],
        compiler_params=pltpu.CompilerParams(
            dimension_semantics=("parallel",)),
    )(v, w, b.reshape(1, NUM_OBJ))


def _rel_head_kernel(e_ref, w_ref, b_ref, f_ref, o_ref):
    d = jnp.dot(e_ref[...], w_ref[...],
                preferred_element_type=jnp.float32) + b_ref[...]
    o_ref[...] = d + f_ref[:, :NUM_REL]


def _rel_head(e, w, b, freq_rows, *, bm=512):
    M = e.shape[0]
    FD = freq_rows.shape[1]
    return pl.pallas_call(
        _rel_head_kernel,
        grid=(M // bm,),
        in_specs=[
            pl.BlockSpec((bm, H), lambda i: (i, 0)),
            pl.BlockSpec((H, NUM_REL), lambda i: (0, 0)),
            pl.BlockSpec((1, NUM_REL), lambda i: (0, 0)),
            pl.BlockSpec((bm, FD), lambda i: (i, 0)),
        ],
        out_specs=pl.BlockSpec((bm, NUM_REL), lambda i: (i, 0)),
        out_shape=jax.ShapeDtypeStruct((M, NUM_REL), jnp.float32),
        compiler_params=pltpu.CompilerParams(
            dimension_semantics=("parallel",)),
    )(e, w, b.reshape(1, NUM_REL), freq_rows)


_NC = 2
_NS = 16
_NW = _NC * _NS


def _sc_gather(table, idx, *, ch=128):
    B = idx.shape[0]
    D = table.shape[1]
    dt = table.dtype
    per_w = B // _NW
    nch = per_w // ch
    mesh = plsc.VectorSubcoreMesh(core_axis_name="c", subcore_axis_name="s", num_cores=_NC, num_subcores=_NS)

    def body(table_hbm, idx_hbm, out_hbm, idx_v, rows_v, sem):
        wid = lax.axis_index("s") * _NC + lax.axis_index("c")
        base = wid * per_w

        def chunk(g, carry):
            off = base + g * ch
            pltpu.sync_copy(idx_hbm.at[pl.ds(off, ch)], idx_v)
            pltpu.async_copy(table_hbm.at[idx_v], rows_v, sem).wait()
            pltpu.sync_copy(rows_v, out_hbm.at[pl.ds(off, ch)])
            return carry

        lax.fori_loop(0, nch, chunk, 0)

    f = pl.kernel(
        body,
        out_type=jax.ShapeDtypeStruct((B, D), dt),
        mesh=mesh,
        scratch_types=[
            pltpu.VMEM((ch,), jnp.int32),
            pltpu.VMEM((ch, D), dt),
            pltpu.SemaphoreType.DMA,
        ],
    )
    return f(table, idx)


_W = 80
_CSTEP = 64
_NR = 128


def _sc_scatter_add(msmo, order_p, sorted_p, starts_p, zeros_acc, n_nodes):
    mesh = plsc.VectorSubcoreMesh(core_axis_name="c", subcore_axis_name="s", num_cores=_NC, num_subcores=_NS)

    def body(m_hbm, ord_hbm, sx_hbm, st_hbm, z_hbm, out_hbm,
             st_v, ord_v, sx_v, rows_v, acc_v, sem):
        wid = lax.axis_index("s") * _NC + lax.axis_index("c")
        base_node = wid * _NR
        pltpu.sync_copy(st_hbm, st_v)
        pltpu.sync_copy(z_hbm, acc_v)
        s16 = st_v[pl.ds(wid, 16)]
        start = s16[0]
        end = s16[1]
        j0 = lax.rem(start, 8)
        p_base = start - j0
        nch = lax.div(end - start + j0 + _CSTEP - 1, _CSTEP)

        def chunk(g, carry):
            p0 = pl.multiple_of(p_base + g * _CSTEP, 8)
            pltpu.sync_copy(ord_hbm.at[pl.ds(p0, _W)], ord_v)
            pltpu.sync_copy(sx_hbm.at[pl.ds(p0, _W)], sx_v)
            pltpu.async_copy(m_hbm.at[ord_v], rows_v, sem).wait()
            lo = p0 + j0
            hi = jnp.minimum(lo + _CSTEP, end)
            for jg in range(_W // 16):
                x16 = sx_v[pl.ds(jg * 16, 16)]
                for j in range(16):
                    jj = jg * 16 + j
                    p = p0 + jj
                    valid = (p >= lo) & (p < hi)
                    nl = jnp.where(valid, x16[j] - base_node, _NR)

                    def col(c4, carry2):
                        for u in range(4):
                            o = c4 * 64 + u * 16
                            plsc.addupdate(acc_v.at[nl, pl.ds(o, 16)],
                                           rows_v[jj, pl.ds(o, 16)])
                        return carry2

                    lax.fori_loop(0, H // 64, col, 0)
            return carry

        lax.fori_loop(0, nch, chunk, 0)
        pltpu.sync_copy(acc_v.at[pl.ds(0, _NR)],
                        out_hbm.at[pl.ds(base_node, _NR)])

    f = pl.kernel(
        body,
        out_type=jax.ShapeDtypeStruct((n_nodes, H), jnp.float32),
        mesh=mesh,
        scratch_types=[
            pltpu.VMEM((48,), jnp.int32),
            pltpu.VMEM((_W,), jnp.int32),
            pltpu.VMEM((_W,), jnp.int32),
            pltpu.VMEM((_W, H), jnp.float32),
            pltpu.VMEM((_NR + 8, H), jnp.float32),
            pltpu.SemaphoreType.DMA,
        ],
    )
    return f(msmo, order_p, sorted_p, starts_p, zeros_acc)


def _pair_kernel(gs_ref, go_ref, o_ref):
    o_ref[...] = gs_ref[:, 0:1] * NUM_OBJ + go_ref[:, 0:1]


def _pair_index(gath_preds, E, *, bm=512):
    nb = E // bm
    return pl.pallas_call(
        _pair_kernel,
        grid=(nb,),
        in_specs=[
            pl.BlockSpec((bm, 128), lambda i: (i, 0)),
            pl.BlockSpec((bm, 128), lambda i: (i + nb, 0)),
        ],
        out_specs=pl.BlockSpec((bm, 1), lambda i: (i, 0)),
        out_shape=jax.ShapeDtypeStruct((E, 1), jnp.int32),
        compiler_params=pltpu.CompilerParams(
            dimension_semantics=("parallel",)),
    )(gath_preds, gath_preds)


def kernel(roi_features, union_features, rel_pair_idx, up_W, up_b, objU_W,
           objU_b, edgeU_W, edgeU_b, node_Wih, node_bih, node_Whh, node_bhh,
           edge_Wih, edge_bih, edge_Whh, edge_bhh, subw_W, subw_b, objw_W,
           objw_b, outw_W, outw_b, inw_W, inw_b, objfc_W, objfc_b, relfc_W,
           relfc_b, freq_table):
    N = roi_features.shape[0]
    E = union_features.shape[0]

    uf = _matmul_bias(union_features, up_W, up_b)
    edge = _mm_gru0(uf, edgeU_W, edgeU_b, edge_Wih, edge_bih, edge_bhh,
                    relu=True, bk=uf.shape[1])
    vert = _mm_gru0(roi_features, objU_W, objU_b, node_Wih, node_bih,
                    node_bhh, relu=False, bk=roi_features.shape[1])

    sub_idx = rel_pair_idx[:, 0]
    obj_idx = rel_pair_idx[:, 1]
    idx_all = jnp.concatenate([sub_idx, obj_idx])

    wa = jnp.concatenate([subw_W, outw_W], axis=1)
    ba = jnp.concatenate([subw_b, outw_b]).reshape(1, 2)
    wb = jnp.concatenate([objw_W, inw_W], axis=1)
    bb = jnp.concatenate([objw_b, inw_b]).reshape(1, 2)

    order = jnp.argsort(idx_all).astype(jnp.int32)
    sorted_idx = idx_all[order]
    starts = jnp.searchsorted(
        sorted_idx, jnp.arange(_NW + 1, dtype=jnp.int32) * _NR).astype(jnp.int32)
    order_p = jnp.concatenate(
        [order, jnp.zeros((160,), jnp.int32)])
    sorted_p = jnp.concatenate(
        [sorted_idx, jnp.zeros((160,), jnp.int32)])
    starts_p = jnp.concatenate(
        [starts, jnp.full((48 - _NW - 1,), 2 * E, jnp.int32)])
    zeros_acc = jnp.zeros((_NR + 8, H), jnp.float32)

    for _ in range(STEPS):
        gath = _sc_gather(vert, idx_all)
        sv = gath[:E]
        ov = gath[E:]
        edge_new, msmo = _edge_step(sv, ov, edge, wa, ba, wb, bb, edge_Wih,
                                    edge_bih, edge_Whh, edge_bhh)
        ctx = _sc_scatter_add(msmo.reshape(2 * E, H), order_p, sorted_p,
                              starts_p, zeros_acc, N)
        vert = _node_gru(ctx, vert, node_Wih, node_bih, node_Whh, node_bhh)
        edge = edge_new

    obj_dists, preds = _obj_head(vert, objfc_W, objfc_b)
    gath_preds = _sc_gather(preds, idx_all)
    pair = _pair_index(gath_preds, E)
    freq_pad = jnp.pad(freq_table, ((0, 0), (0, 128 - NUM_REL)))
    freq_rows = _sc_gather(freq_pad, pair.reshape(-1))
    rel_dists = _rel_head(edge, relfc_W, relfc_b, freq_rows)
    return (obj_dists, rel_dists)


# speedup vs baseline: 1.1463x; 1.0066x over previous
import functools

import jax
import jax.numpy as jnp
from jax import lax
from jax.experimental import pallas as pl
from jax.experimental.pallas import tpu as pltpu
from jax.experimental.pallas import tpu_sc as plsc

H = 512
NUM_OBJ = 151
NUM_REL = 51
STEPS = 3


def _mm_kernel(a_ref, b_ref, bias_ref, o_ref):
    o_ref[...] = jnp.dot(a_ref[...], b_ref[...],
                         preferred_element_type=jnp.float32) + bias_ref[...]


def _matmul_bias(a, b, bias, *, bm=256, bn=512):
    M, K = a.shape
    _, N = b.shape
    return pl.pallas_call(
        _mm_kernel,
        grid=(M // bm, N // bn),
        in_specs=[
            pl.BlockSpec((bm, K), lambda i, j: (i, 0)),
            pl.BlockSpec((K, bn), lambda i, j: (0, j)),
            pl.BlockSpec((1, bn), lambda i, j: (0, j)),
        ],
        out_specs=pl.BlockSpec((bm, bn), lambda i, j: (i, j)),
        out_shape=jax.ShapeDtypeStruct((M, N), jnp.float32),
        compiler_params=pltpu.CompilerParams(
            dimension_semantics=("parallel", "parallel")),
    )(a, b, bias.reshape(1, N))


def _gru0_tail(acc, b1, relu, wih_ref, bih_ref, bhh_ref, o_ref):
    rep = acc + b1
    if relu:
        rep = jnp.maximum(rep, 0.0)
    gi = jnp.dot(rep, wih_ref[...],
                 preferred_element_type=jnp.float32) + bih_ref[...]
    bhh = bhh_ref[...]
    r = jax.nn.sigmoid(gi[:, :H] + bhh[:, :H])
    z = jax.nn.sigmoid(gi[:, H:2 * H] + bhh[:, H:2 * H])
    n = jnp.tanh(gi[:, 2 * H:] + r * bhh[:, 2 * H:])
    o_ref[...] = (1.0 - z) * n


def _mm_gru0_kernel(a_ref, w1_ref, b1_ref, wih_ref, bih_ref, bhh_ref, o_ref,
                    acc_ref, *, nk, relu):
    k = pl.program_id(1)

    @pl.when(k == 0)
    def _():
        acc_ref[...] = jnp.zeros_like(acc_ref)

    acc_ref[...] += jnp.dot(a_ref[...], w1_ref[...],
                            preferred_element_type=jnp.float32)

    @pl.when(k == nk - 1)
    def _():
        _gru0_tail(acc_ref[...], b1_ref[...], relu, wih_ref, bih_ref,
                   bhh_ref, o_ref)


def _mm_gru0(a, w1, b1, wih, bih, bhh, *, relu, bm=256, bk=512):
    M, K = a.shape
    nk = K // bk
    return pl.pallas_call(
        functools.partial(_mm_gru0_kernel, nk=nk, relu=relu),
        grid=(M // bm, nk),
        in_specs=[
            pl.BlockSpec((bm, bk), lambda i, k: (i, k)),
            pl.BlockSpec((bk, H), lambda i, k: (k, 0)),
            pl.BlockSpec((1, H), lambda i, k: (0, 0)),
            pl.BlockSpec((H, 3 * H), lambda i, k: (0, 0)),
            pl.BlockSpec((1, 3 * H), lambda i, k: (0, 0)),
            pl.BlockSpec((1, 3 * H), lambda i, k: (0, 0)),
        ],
        out_specs=pl.BlockSpec((bm, H), lambda i, k: (i, 0)),
        out_shape=jax.ShapeDtypeStruct((M, H), jnp.float32),
        scratch_shapes=[pltpu.VMEM((bm, H), jnp.float32)],
        compiler_params=pltpu.CompilerParams(
            dimension_semantics=("parallel", "arbitrary")),
    )(a, w1, b1.reshape(1, H), wih, bih.reshape(1, 3 * H),
      bhh.reshape(1, 3 * H))


def _edge_step_kernel(sv_ref, ov_ref, e_ref, wa_ref, ba_ref, wb_ref, bb_ref,
                      wih_ref, bih_ref, whh_ref, bhh_ref, ne_ref, msmo_ref):
    sv = sv_ref[...]
    ov = ov_ref[...]
    e = e_ref[...]
    wa = wa_ref[...]
    wb = wb_ref[...]
    xa = jnp.concatenate([sv, e], axis=1)
    xb = jnp.concatenate([ov, e], axis=1)
    ga = jnp.dot(xa, wa, preferred_element_type=jnp.float32) + ba_ref[...]
    gb = jnp.dot(xb, wb, preferred_element_type=jnp.float32) + bb_ref[...]
    ws = jax.nn.sigmoid(ga[:, 0:1])
    w_out = jax.nn.sigmoid(ga[:, 1:2])
    wo = jax.nn.sigmoid(gb[:, 0:1])
    w_in = jax.nn.sigmoid(gb[:, 1:2])
    x = ws * sv + wo * ov
    gi = jnp.dot(x, wih_ref[...],
                 preferred_element_type=jnp.float32) + bih_ref[...]
    gh = jnp.dot(e, whh_ref[...],
                 preferred_element_type=jnp.float32) + bhh_ref[...]
    r = jax.nn.sigmoid(gi[:, :H] + gh[:, :H])
    z = jax.nn.sigmoid(gi[:, H:2 * H] + gh[:, H:2 * H])
    n = jnp.tanh(gi[:, 2 * H:] + r * gh[:, 2 * H:])
    ne_ref[...] = (1.0 - z) * n + z * e
    msmo_ref[0] = w_out * e
    msmo_ref[1] = w_in * e


def _edge_step(sv, ov, e, wa, ba, wb, bb, wih, bih, whh, bhh, *, bm=256):
    E = e.shape[0]
    row = lambda i: (i, 0)
    cst = lambda i: (0, 0)
    out_sds = jax.ShapeDtypeStruct((E, H), jnp.float32)
    return pl.pallas_call(
        _edge_step_kernel,
        grid=(E // bm,),
        in_specs=[
            pl.BlockSpec((bm, H), row),
            pl.BlockSpec((bm, H), row),
            pl.BlockSpec((bm, H), row),
            pl.BlockSpec((2 * H, 2), cst),
            pl.BlockSpec((1, 2), cst),
            pl.BlockSpec((2 * H, 2), cst),
            pl.BlockSpec((1, 2), cst),
            pl.BlockSpec((H, 3 * H), cst),
            pl.BlockSpec((1, 3 * H), cst),
            pl.BlockSpec((H, 3 * H), cst),
            pl.BlockSpec((1, 3 * H), cst),
        ],
        out_specs=[
            pl.BlockSpec((bm, H), row),
            pl.BlockSpec((2, bm, H), lambda i: (0, i, 0)),
        ],
        out_shape=[out_sds,
                   jax.ShapeDtypeStruct((2, E, H), jnp.float32)],
        compiler_params=pltpu.CompilerParams(
            dimension_semantics=("parallel",)),
    )(sv, ov, e, wa, ba, wb, bb, wih, bih.reshape(1, 3 * H), whh,
      bhh.reshape(1, 3 * H))


def _node_gru_kernel(x_ref, h_ref, wih_ref, bih_ref, whh_ref, bhh_ref, o_ref):
    h = h_ref[...]
    gi = jnp.dot(x_ref[...], wih_ref[...],
                 preferred_element_type=jnp.float32) + bih_ref[...]
    gh = jnp.dot(h, whh_ref[...],
                 preferred_element_type=jnp.float32) + bhh_ref[...]
    r = jax.nn.sigmoid(gi[:, :H] + gh[:, :H])
    z = jax.nn.sigmoid(gi[:, H:2 * H] + gh[:, H:2 * H])
    n = jnp.tanh(gi[:, 2 * H:] + r * gh[:, 2 * H:])
    o_ref[...] = (1.0 - z) * n + z * h


def _node_gru(x, h, wih, bih, whh, bhh, *, bm=256):
    M = x.shape[0]
    row = lambda i: (i, 0)
    cst = lambda i: (0, 0)
    return pl.pallas_call(
        _node_gru_kernel,
        grid=(M // bm,),
        in_specs=[
            pl.BlockSpec((bm, H), row),
            pl.BlockSpec((bm, H), row),
            pl.BlockSpec((H, 3 * H), cst),
            pl.BlockSpec((1, 3 * H), cst),
            pl.BlockSpec((H, 3 * H), cst),
            pl.BlockSpec((1, 3 * H), cst),
        ],
        out_specs=pl.BlockSpec((bm, H), row),
        out_shape=jax.ShapeDtypeStruct((M, H), jnp.float32),
        compiler_params=pltpu.CompilerParams(
            dimension_semantics=("parallel",)),
    )(x, h, wih, bih.reshape(1, 3 * H), whh, bhh.reshape(1, 3 * H))


def _obj_head_kernel(v_ref, w_ref, b_ref, d_ref, p_ref):
    d = jnp.dot(v_ref[...], w_ref[...],
                preferred_element_type=jnp.float32) + b_ref[...]
    d_ref[...] = d
    m = jnp.max(d, axis=1, keepdims=True)
    idx = lax.broadcasted_iota(jnp.int32, d.shape, 1)
    p = jnp.min(jnp.where(d == m, idx, NUM_OBJ), axis=1, keepdims=True)
    p_ref[...] = jnp.broadcast_to(p, p_ref.shape)


def _obj_head(v, w, b, *, bm=512):
    M = v.shape[0]
    return pl.pallas_call(
        _obj_head_kernel,
        grid=(M // bm,),
        in_specs=[
            pl.BlockSpec((bm, H), lambda i: (i, 0)),
            pl.BlockSpec((H, NUM_OBJ), lambda i: (0, 0)),
            pl.BlockSpec((1, NUM_OBJ), lambda i: (0, 0)),
        ],
        out_specs=[
            pl.BlockSpec((bm, NUM_OBJ), lambda i: (i, 0)),
            pl.BlockSpec((bm, 128), lambda i: (i, 0)),
        ],
        out_shape=[
            jax.ShapeDtypeStruct((M, NUM_OBJ), jnp.float32),
            jax.ShapeDtypeStruct((M, 128), jnp.int32),
        ],
        compiler_params=pltpu.CompilerParams(
            dimension_semantics=("parallel",)),
    )(v, w, b.reshape(1, NUM_OBJ))


def _rel_head_kernel(e_ref, w_ref, b_ref, f_ref, o_ref):
    d = jnp.dot(e_ref[...], w_ref[...],
                preferred_element_type=jnp.float32) + b_ref[...]
    o_ref[...] = d + f_ref[:, :NUM_REL]


def _rel_head(e, w, b, freq_rows, *, bm=512):
    M = e.shape[0]
    FD = freq_rows.shape[1]
    return pl.pallas_call(
        _rel_head_kernel,
        grid=(M // bm,),
        in_specs=[
            pl.BlockSpec((bm, H), lambda i: (i, 0)),
            pl.BlockSpec((H, NUM_REL), lambda i: (0, 0)),
            pl.BlockSpec((1, NUM_REL), lambda i: (0, 0)),
            pl.BlockSpec((bm, FD), lambda i: (i, 0)),
        ],
        out_specs=pl.BlockSpec((bm, NUM_REL), lambda i: (i, 0)),
        out_shape=jax.ShapeDtypeStruct((M, NUM_REL), jnp.float32),
        compiler_params=pltpu.CompilerParams(
            dimension_semantics=("parallel",)),
    )(e, w, b.reshape(1, NUM_REL), freq_rows)


_NC = 2
_NS = 16
_NW = _NC * _NS


def _sc_gather(table, idx, *, ch=64):
    B = idx.shape[0]
    D = table.shape[1]
    dt = table.dtype
    per_w = B // _NW
    nch = per_w // ch
    mesh = plsc.VectorSubcoreMesh(core_axis_name="c", subcore_axis_name="s", num_cores=_NC, num_subcores=_NS)

    def body(table_hbm, idx_hbm, out_hbm, idx_v, rows0, rows1, sem0, sem1):
        wid = lax.axis_index("s") * _NC + lax.axis_index("c")
        base = wid * per_w
        pltpu.sync_copy(idx_hbm.at[pl.ds(base, per_w)], idx_v)

        def gth(g, buf, sem):
            pltpu.async_copy(table_hbm.at[idx_v.at[pl.ds(g * ch, ch)]],
                             buf, sem)

        def wt(buf, sem):
            pltpu.make_async_copy(
                table_hbm.at[idx_v.at[pl.ds(0, ch)]], buf, sem).wait()

        gth(0, rows0, sem0)

        def pair(g2, carry):
            g = g2 * 2
            gth(g + 1, rows1, sem1)
            wt(rows0, sem0)
            pltpu.sync_copy(rows0, out_hbm.at[pl.ds(base + g * ch, ch)])

            @pl.when(g + 2 < nch)
            def _():
                gth(g + 2, rows0, sem0)

            wt(rows1, sem1)
            pltpu.sync_copy(rows1, out_hbm.at[pl.ds(base + (g + 1) * ch, ch)])
            return carry

        lax.fori_loop(0, nch // 2, pair, 0)

    f = pl.kernel(
        body,
        out_type=jax.ShapeDtypeStruct((B, D), dt),
        mesh=mesh,
        scratch_types=[
            pltpu.VMEM((per_w,), jnp.int32),
            pltpu.VMEM((ch, D), dt),
            pltpu.VMEM((ch, D), dt),
            pltpu.SemaphoreType.DMA,
            pltpu.SemaphoreType.DMA,
        ],
    )
    return f(table, idx)


_W = 80
_CSTEP = 64
_NR = 128


def _sc_scatter_add(msmo, order_p, sorted_p, starts_p, zeros_acc, n_nodes):
    mesh = plsc.VectorSubcoreMesh(core_axis_name="c", subcore_axis_name="s", num_cores=_NC, num_subcores=_NS)

    def body(m_hbm, ord_hbm, sx_hbm, st_hbm, z_hbm, out_hbm,
             st_v, ord_v, sx_v, rows_v, acc_v, sem):
        wid = lax.axis_index("s") * _NC + lax.axis_index("c")
        base_node = wid * _NR
        pltpu.sync_copy(st_hbm, st_v)
        pltpu.sync_copy(z_hbm, acc_v)
        s16 = st_v[pl.ds(wid, 16)]
        start = s16[0]
        end = s16[1]
        j0 = lax.rem(start, 8)
        p_base = start - j0
        nch = lax.div(end - start + j0 + _CSTEP - 1, _CSTEP)

        def chunk(g, carry):
            p0 = pl.multiple_of(p_base + g * _CSTEP, 8)
            pltpu.sync_copy(ord_hbm.at[pl.ds(p0, _W)], ord_v)
            pltpu.sync_copy(sx_hbm.at[pl.ds(p0, _W)], sx_v)
            pltpu.async_copy(m_hbm.at[ord_v], rows_v, sem).wait()
            lo = p0 + j0
            hi = jnp.minimum(lo + _CSTEP, end)
            for jg in range(_W // 16):
                x16 = sx_v[pl.ds(jg * 16, 16)]
                for j in range(16):
                    jj = jg * 16 + j
                    p = p0 + jj
                    valid = (p >= lo) & (p < hi)
                    nl = jnp.where(valid, x16[j] - base_node, _NR)

                    def col(c4, carry2):
                        for u in range(4):
                            o = c4 * 64 + u * 16
                            plsc.addupdate(acc_v.at[nl, pl.ds(o, 16)],
                                           rows_v[jj, pl.ds(o, 16)])
                        return carry2

                    lax.fori_loop(0, H // 64, col, 0)
            return carry

        lax.fori_loop(0, nch, chunk, 0)
        pltpu.sync_copy(acc_v.at[pl.ds(0, _NR)],
                        out_hbm.at[pl.ds(base_node, _NR)])

    f = pl.kernel(
        body,
        out_type=jax.ShapeDtypeStruct((n_nodes, H), jnp.float32),
        mesh=mesh,
        scratch_types=[
            pltpu.VMEM((48,), jnp.int32),
            pltpu.VMEM((_W,), jnp.int32),
            pltpu.VMEM((_W,), jnp.int32),
            pltpu.VMEM((_W, H), jnp.float32),
            pltpu.VMEM((_NR + 8, H), jnp.float32),
            pltpu.SemaphoreType.DMA,
        ],
    )
    return f(msmo, order_p, sorted_p, starts_p, zeros_acc)


def _pair_kernel(gs_ref, go_ref, o_ref):
    o_ref[...] = gs_ref[:, 0:1] * NUM_OBJ + go_ref[:, 0:1]


def _pair_index(gath_preds, E, *, bm=512):
    nb = E // bm
    return pl.pallas_call(
        _pair_kernel,
        grid=(nb,),
        in_specs=[
            pl.BlockSpec((bm, 128), lambda i: (i, 0)),
            pl.BlockSpec((bm, 128), lambda i: (i + nb, 0)),
        ],
        out_specs=pl.BlockSpec((bm, 1), lambda i: (i, 0)),
        out_shape=jax.ShapeDtypeStruct((E, 1), jnp.int32),
        compiler_params=pltpu.CompilerParams(
            dimension_semantics=("parallel",)),
    )(gath_preds, gath_preds)


def kernel(roi_features, union_features, rel_pair_idx, up_W, up_b, objU_W,
           objU_b, edgeU_W, edgeU_b, node_Wih, node_bih, node_Whh, node_bhh,
           edge_Wih, edge_bih, edge_Whh, edge_bhh, subw_W, subw_b, objw_W,
           objw_b, outw_W, outw_b, inw_W, inw_b, objfc_W, objfc_b, relfc_W,
           relfc_b, freq_table):
    N = roi_features.shape[0]
    E = union_features.shape[0]

    uf = _matmul_bias(union_features, up_W, up_b)
    edge = _mm_gru0(uf, edgeU_W, edgeU_b, edge_Wih, edge_bih, edge_bhh,
                    relu=True, bk=uf.shape[1])
    vert = _mm_gru0(roi_features, objU_W, objU_b, node_Wih, node_bih,
                    node_bhh, relu=False, bk=roi_features.shape[1])

    sub_idx = rel_pair_idx[:, 0]
    obj_idx = rel_pair_idx[:, 1]
    idx_all = jnp.concatenate([sub_idx, obj_idx])

    wa = jnp.concatenate([subw_W, outw_W], axis=1)
    ba = jnp.concatenate([subw_b, outw_b]).reshape(1, 2)
    wb = jnp.concatenate([objw_W, inw_W], axis=1)
    bb = jnp.concatenate([objw_b, inw_b]).reshape(1, 2)

    order = jnp.argsort(idx_all).astype(jnp.int32)
    sorted_idx = idx_all[order]
    starts = jnp.searchsorted(
        sorted_idx, jnp.arange(_NW + 1, dtype=jnp.int32) * _NR).astype(jnp.int32)
    order_p = jnp.concatenate(
        [order, jnp.zeros((160,), jnp.int32)])
    sorted_p = jnp.concatenate(
        [sorted_idx, jnp.zeros((160,), jnp.int32)])
    starts_p = jnp.concatenate(
        [starts, jnp.full((48 - _NW - 1,), 2 * E, jnp.int32)])
    zeros_acc = jnp.zeros((_NR + 8, H), jnp.float32)

    for _ in range(STEPS):
        gath = _sc_gather(vert, idx_all)
        sv = gath[:E]
        ov = gath[E:]
        edge_new, msmo = _edge_step(sv, ov, edge, wa, ba, wb, bb, edge_Wih,
                                    edge_bih, edge_Whh, edge_bhh)
        ctx = _sc_scatter_add(msmo.reshape(2 * E, H), order_p, sorted_p,
                              starts_p, zeros_acc, N)
        vert = _node_gru(ctx, vert, node_Wih, node_bih, node_Whh, node_bhh)
        edge = edge_new

    obj_dists, preds = _obj_head(vert, objfc_W, objfc_b)
    gath_preds = _sc_gather(preds, idx_all)
    pair = _pair_index(gath_preds, E)
    freq_pad = jnp.pad(freq_table, ((0, 0), (0, 128 - NUM_REL)))
    freq_rows = _sc_gather(freq_pad, pair.reshape(-1))
    rel_dists = _rel_head(edge, relfc_W, relfc_b, freq_rows)
    return (obj_dists, rel_dists)
